# Initial kernel scaffold; baseline (speedup 1.0000x reference)
import jax
import jax.numpy as jnp
from jax.experimental import pallas as pl


def kernel(img, spx):
    raise NotImplementedError("write your pallas kernel here")


# speedup vs baseline: 3.8793x; 3.8793x over previous
import functools

import jax
import jax.numpy as jnp
from jax import lax
from jax.experimental import pallas as pl
from jax.experimental.pallas import tpu as pltpu
from jax.experimental.pallas import tpu_sc as plsc

_L = 16
_PC = 2048
_NBUF = 2


def _build_sc_call(B, C, P, K):
    info = plsc.get_sparse_core_info()
    NC, NS = info.num_cores, info.num_subcores
    NW = NC * NS
    assert (B * C) % NW == 0
    CPT = (B * C) // NW
    TPB = NW // B
    assert C == CPT * TPB
    assert P % _PC == 0
    NCHUNK = P // _PC
    assert NCHUNK % _NBUF == 0
    NV = _PC // _L

    mesh = plsc.VectorSubcoreMesh(core_axis_name="c", subcore_axis_name="s")

    @functools.partial(
        pl.kernel,
        out_type=jax.ShapeDtypeStruct((B * C * K,), jnp.float32),
        mesh=mesh,
        scratch_types=[
            pltpu.VMEM((_NBUF, _PC), jnp.int32),
            pltpu.VMEM((_NBUF, CPT, _PC), jnp.float32),
            pltpu.VMEM((CPT * K,), jnp.float32),
            pltpu.VMEM((K,), jnp.float32),
            pltpu.SemaphoreType.DMA,
            pltpu.SemaphoreType.DMA,
        ],
        compiler_params=pltpu.CompilerParams(
            use_tc_tiling_on_sc=False, needs_layout_passes=False),
    )
    def sc_pool(img_hbm, spx_hbm, out_hbm, idx_buf, data_buf, acc, cnt,
                sem0, sem1):
        wid = lax.axis_index("s") * NC + lax.axis_index("c")
        b = wid // TPB
        cg = wid % TPB
        row0 = b * C + cg * CPT

        sems = (sem0, sem1)
        zero16 = jnp.zeros((_L,), jnp.float32)
        ones16 = jnp.ones((_L,), jnp.float32)

        @pl.loop(0, (CPT * K) // _L)
        def _zero_acc(v):
            acc[pl.ds(v * _L, _L)] = zero16

        @pl.loop(0, K // _L)
        def _zero_cnt(v):
            cnt[pl.ds(v * _L, _L)] = zero16

        def issue(ch, buf):
            start = ch * _PC
            pltpu.async_copy(spx_hbm.at[pl.ds(b * P + start, _PC)],
                             idx_buf.at[buf], sems[buf])
            for cc in range(CPT):
                pltpu.async_copy(
                    img_hbm.at[pl.ds((row0 + cc) * P + start, _PC)],
                    data_buf.at[buf, cc], sems[buf])

        def wait(buf):
            pltpu.make_async_copy(spx_hbm.at[pl.ds(0, _PC)],
                                  idx_buf.at[buf], sems[buf]).wait()
            for cc in range(CPT):
                pltpu.make_async_copy(img_hbm.at[pl.ds(0, _PC)],
                                      data_buf.at[buf, cc],
                                      sems[buf]).wait()

        issue(0, 0)

        @pl.loop(0, NCHUNK // _NBUF)
        def _chunk_group(g):
            for buf in range(_NBUF):
                ch = g * _NBUF + buf
                wait(buf)

                @pl.when(ch + 1 < NCHUNK)
                def _prefetch():
                    issue(ch + 1, 1 - buf)

                @pl.loop(0, NV)
                def _vec(v):
                    base = v * _L
                    iv = idx_buf[buf, pl.ds(base, _L)]
                    plsc.addupdate_scatter(cnt, [iv], ones16)
                    for cc in range(CPT):
                        x = data_buf[buf, cc, pl.ds(base, _L)]
                        plsc.addupdate_scatter(acc, [iv + cc * K], x)

        @pl.loop(0, K // _L)
        def _finalize(v):
            base = v * _L
            r = 1.0 / jnp.maximum(cnt[pl.ds(base, _L)], 1.0)
            for cc in range(CPT):
                s = pl.ds(cc * K + base, _L)
                acc[s] = acc[s] * r

        pltpu.sync_copy(acc, out_hbm.at[pl.ds(row0 * K, CPT * K)])

    return sc_pool


def kernel(img, spx):
    B, C, H, W = img.shape
    P = H * W
    K = 1024
    img2 = img.reshape(B * C * P)
    spx2 = spx.reshape(B * P)
    out = _build_sc_call(B, C, P, K)(img2, spx2)
    return out.reshape(B, C, K)


# speedup vs baseline: 6.2533x; 1.6119x over previous
import functools

import jax
import jax.numpy as jnp
from jax import lax
from jax.experimental import pallas as pl
from jax.experimental.pallas import tpu as pltpu
from jax.experimental.pallas import tpu_sc as plsc

_L = 16
_PC = 2048
_NBUF = 2


def _build_sc_call(B, C, P, K):
    info = plsc.get_sparse_core_info()
    NC, NS = info.num_cores, info.num_subcores
    NW = NC * NS
    assert (B * C) % NW == 0
    CPT = (B * C) // NW
    TPB = NW // B
    assert C == CPT * TPB
    assert P % _PC == 0
    NCHUNK = P // _PC
    assert NCHUNK % _NBUF == 0
    NV = _PC // _L

    mesh = plsc.VectorSubcoreMesh(core_axis_name="c", subcore_axis_name="s")

    @functools.partial(
        pl.kernel,
        out_type=jax.ShapeDtypeStruct((B * C * K,), jnp.float32),
        mesh=mesh,
        scratch_types=[
            pltpu.VMEM((_NBUF, _PC), jnp.int32),
            pltpu.VMEM((_NBUF, CPT, _PC), jnp.float32),
            pltpu.VMEM((CPT * K,), jnp.float32),
            pltpu.VMEM((K,), jnp.float32),
            pltpu.SemaphoreType.DMA,
            pltpu.SemaphoreType.DMA,
        ],
        compiler_params=pltpu.CompilerParams(
            use_tc_tiling_on_sc=False, needs_layout_passes=False),
    )
    def sc_pool(img_hbm, spx_hbm, out_hbm, idx_buf, data_buf, acc, cnt,
                sem0, sem1):
        wid = lax.axis_index("s") * NC + lax.axis_index("c")
        b = wid // TPB
        cg = wid % TPB
        row0 = b * C + cg * CPT

        sems = (sem0, sem1)
        zero16 = jnp.zeros((_L,), jnp.float32)
        ones16 = jnp.ones((_L,), jnp.float32)

        @plsc.parallel_loop(0, (CPT * K) // _L, unroll=4)
        def _zero_acc(v):
            acc[pl.ds(v * _L, _L)] = zero16

        @plsc.parallel_loop(0, K // _L, unroll=4)
        def _zero_cnt(v):
            cnt[pl.ds(v * _L, _L)] = zero16

        def issue(ch, buf):
            start = ch * _PC
            pltpu.async_copy(spx_hbm.at[pl.ds(b * P + start, _PC)],
                             idx_buf.at[buf], sems[buf])
            for cc in range(CPT):
                pltpu.async_copy(
                    img_hbm.at[pl.ds((row0 + cc) * P + start, _PC)],
                    data_buf.at[buf, cc], sems[buf])

        def wait(buf):
            pltpu.make_async_copy(spx_hbm.at[pl.ds(0, _PC)],
                                  idx_buf.at[buf], sems[buf]).wait()
            for cc in range(CPT):
                pltpu.make_async_copy(img_hbm.at[pl.ds(0, _PC)],
                                      data_buf.at[buf, cc],
                                      sems[buf]).wait()

        issue(0, 0)

        @pl.loop(0, NCHUNK // _NBUF)
        def _chunk_group(g):
            for buf in range(_NBUF):
                ch = g * _NBUF + buf
                wait(buf)

                @pl.when(ch + 1 < NCHUNK)
                def _prefetch():
                    issue(ch + 1, 1 - buf)

                @plsc.parallel_loop(0, NV, unroll=2)
                def _vec(v):
                    base = v * _L
                    iv = idx_buf[buf, pl.ds(base, _L)]
                    plsc.addupdate_scatter(cnt, [iv], ones16)
                    for cc in range(CPT):
                        x = data_buf[buf, cc, pl.ds(base, _L)]
                        plsc.addupdate_scatter(acc, [iv + cc * K], x)

        @plsc.parallel_loop(0, K // _L, unroll=2)
        def _finalize(v):
            base = v * _L
            r = 1.0 / jnp.maximum(cnt[pl.ds(base, _L)], 1.0)
            for cc in range(CPT):
                s = pl.ds(cc * K + base, _L)
                acc[s] = acc[s] * r

        pltpu.sync_copy(acc, out_hbm.at[pl.ds(row0 * K, CPT * K)])

    return sc_pool


def kernel(img, spx):
    B, C, H, W = img.shape
    P = H * W
    K = 1024
    img2 = img.reshape(B * C * P)
    spx2 = spx.reshape(B * P)
    out = _build_sc_call(B, C, P, K)(img2, spx2)
    return out.reshape(B, C, K)


# speedup vs baseline: 6.3096x; 1.0090x over previous
import functools

import jax
import jax.numpy as jnp
from jax import lax
from jax.experimental import pallas as pl
from jax.experimental.pallas import tpu as pltpu
from jax.experimental.pallas import tpu_sc as plsc

_L = 16
_PC = 2048
_NBUF = 2


def _build_sc_call(B, C, P, K):
    info = plsc.get_sparse_core_info()
    NC, NS = info.num_cores, info.num_subcores
    NW = NC * NS
    assert (B * C) % NW == 0
    CPT = (B * C) // NW
    TPB = NW // B
    assert C == CPT * TPB
    assert P % _PC == 0
    NCHUNK = P // _PC
    assert NCHUNK % _NBUF == 0
    NV = _PC // _L

    mesh = plsc.VectorSubcoreMesh(core_axis_name="c", subcore_axis_name="s")

    @functools.partial(
        pl.kernel,
        out_type=jax.ShapeDtypeStruct((B * C * K,), jnp.float32),
        mesh=mesh,
        scratch_types=[
            pltpu.VMEM((_NBUF, _PC // _L, _L), jnp.int32),
            pltpu.VMEM((_NBUF, CPT, _PC // _L, _L), jnp.float32),
        ] + [pltpu.VMEM((K,), jnp.float32) for _ in range(12 + 1)] + [
            pltpu.SemaphoreType.DMA,
            pltpu.SemaphoreType.DMA,
        ],
        compiler_params=pltpu.CompilerParams(
            use_tc_tiling_on_sc=False, needs_layout_passes=False),
    )
    def sc_pool(img_hbm, spx_hbm, out_hbm, idx_buf, data_buf, *rest):
        accs = rest[:CPT]
        cnt = rest[CPT]
        sem0, sem1 = rest[CPT + 1], rest[CPT + 2]
        wid = lax.axis_index("s") * NC + lax.axis_index("c")
        b = wid // TPB
        cg = wid % TPB
        row0 = b * C + cg * CPT

        sems = (sem0, sem1)
        zero16 = jnp.zeros((_L,), jnp.float32)
        ones16 = jnp.ones((_L,), jnp.float32)

        @plsc.parallel_loop(0, K // _L, unroll=4)
        def _zero_acc(v):
            for cc in range(CPT):
                accs[cc][pl.ds(v * _L, _L)] = zero16

        @plsc.parallel_loop(0, K // _L, unroll=4)
        def _zero_cnt(v):
            cnt[pl.ds(v * _L, _L)] = zero16

        NG = _PC // _L

        def chunk_of(i):
            return lax.rem(i + cg * (NCHUNK // TPB), NCHUNK)

        def issue(ch, buf):
            g0 = ch * NG
            pltpu.async_copy(spx_hbm.at[b, pl.ds(g0, NG), :],
                             idx_buf.at[buf], sems[buf])
            pltpu.async_copy(img_hbm.at[pl.ds(row0, CPT), pl.ds(g0, NG), :],
                             data_buf.at[buf], sems[buf])

        def wait(buf):
            pltpu.make_async_copy(spx_hbm.at[b, pl.ds(0, NG), :],
                                  idx_buf.at[buf], sems[buf]).wait()
            pltpu.make_async_copy(img_hbm.at[pl.ds(row0, CPT), pl.ds(0, NG), :],
                                  data_buf.at[buf], sems[buf]).wait()

        issue(chunk_of(0), 0)

        @pl.loop(0, NCHUNK // _NBUF)
        def _chunk_group(g):
            for buf in range(_NBUF):
                i = g * _NBUF + buf
                wait(buf)

                @pl.when(i + 1 < NCHUNK)
                def _prefetch():
                    issue(chunk_of(i + 1), 1 - buf)

                @plsc.parallel_loop(0, NV, unroll=2)
                def _vec(v):
                    iv = idx_buf[buf, v]
                    plsc.addupdate_scatter(cnt, [iv], ones16)
                    for cc in range(CPT):
                        x = data_buf[buf, cc, v]
                        plsc.addupdate_scatter(accs[cc], [iv], x)

        @plsc.parallel_loop(0, K // _L, unroll=2)
        def _finalize(v):
            base = v * _L
            r = 1.0 / jnp.maximum(cnt[pl.ds(base, _L)], 1.0)
            for cc in range(CPT):
                accs[cc][pl.ds(base, _L)] = accs[cc][pl.ds(base, _L)] * r

        for cc in range(CPT):
            pltpu.sync_copy(accs[cc],
                            out_hbm.at[pl.ds((row0 + cc) * K, K)])

    return sc_pool


def kernel(img, spx):
    B, C, H, W = img.shape
    P = H * W
    K = 1024
    img2 = img.reshape(B * C, P // 16, 16)
    spx2 = spx.reshape(B, P // 16, 16)
    out = _build_sc_call(B, C, P, K)(img2, spx2)
    return out.reshape(B, C, K)


# speedup vs baseline: 6.3692x; 1.0094x over previous
import functools

import jax
import jax.numpy as jnp
from jax import lax
from jax.experimental import pallas as pl
from jax.experimental.pallas import tpu as pltpu
from jax.experimental.pallas import tpu_sc as plsc

_L = 16
_PC = 2048
_NBUF = 2


def _build_sc_call(B, C, P, K):
    info = plsc.get_sparse_core_info()
    NC, NS = info.num_cores, info.num_subcores
    NW = NC * NS
    assert (B * C) % NW == 0
    CPT = (B * C) // NW
    TPB = NW // B
    assert C == CPT * TPB
    assert P % _PC == 0
    NCHUNK = P // _PC
    assert NCHUNK % _NBUF == 0
    NV = _PC // _L

    mesh = plsc.VectorSubcoreMesh(core_axis_name="c", subcore_axis_name="s")

    @functools.partial(
        pl.kernel,
        out_type=jax.ShapeDtypeStruct((B * C * K,), jnp.float32),
        mesh=mesh,
        scratch_types=[
            pltpu.VMEM((_NBUF, _PC // _L, _L), jnp.int32),
            pltpu.VMEM((_NBUF, CPT, _PC // _L, _L), jnp.float32),
        ] + [pltpu.VMEM((4 * K,), jnp.float32) for _ in range(12 + 1)] + [
            pltpu.SemaphoreType.DMA,
            pltpu.SemaphoreType.DMA,
        ],
        compiler_params=pltpu.CompilerParams(
            use_tc_tiling_on_sc=False, needs_layout_passes=False),
    )
    def sc_pool(img_hbm, spx_hbm, out_hbm, idx_buf, data_buf, *rest):
        accs = rest[:CPT]
        cnt = rest[CPT]
        sem0, sem1 = rest[CPT + 1], rest[CPT + 2]
        wid = lax.axis_index("s") * NC + lax.axis_index("c")
        b = wid // TPB
        cg = wid % TPB
        row0 = b * C + cg * CPT

        sems = (sem0, sem1)
        zero16 = jnp.zeros((_L,), jnp.float32)
        ones16 = jnp.ones((_L,), jnp.float32)

        @plsc.parallel_loop(0, (4 * K) // _L, unroll=4)
        def _zero_acc(v):
            for cc in range(CPT):
                accs[cc][pl.ds(v * _L, _L)] = zero16
            cnt[pl.ds(v * _L, _L)] = zero16

        lane4 = lax.iota(jnp.int32, _L) & jnp.int32(3)

        NG = _PC // _L

        def chunk_of(i):
            return lax.rem(i + cg * (NCHUNK // TPB), NCHUNK)

        def issue(ch, buf):
            g0 = ch * NG
            pltpu.async_copy(spx_hbm.at[b, pl.ds(g0, NG), :],
                             idx_buf.at[buf], sems[buf])
            pltpu.async_copy(img_hbm.at[pl.ds(row0, CPT), pl.ds(g0, NG), :],
                             data_buf.at[buf], sems[buf])

        def wait(buf):
            pltpu.make_async_copy(spx_hbm.at[b, pl.ds(0, NG), :],
                                  idx_buf.at[buf], sems[buf]).wait()
            pltpu.make_async_copy(img_hbm.at[pl.ds(row0, CPT), pl.ds(0, NG), :],
                                  data_buf.at[buf], sems[buf]).wait()

        issue(chunk_of(0), 0)

        @pl.loop(0, NCHUNK // _NBUF)
        def _chunk_group(g):
            for buf in range(_NBUF):
                i = g * _NBUF + buf
                wait(buf)

                @pl.when(i + 1 < NCHUNK)
                def _prefetch():
                    issue(chunk_of(i + 1), 1 - buf)

                @plsc.parallel_loop(0, NV, unroll=2)
                def _vec(v):
                    iv = (idx_buf[buf, v] << 2) | lane4
                    plsc.addupdate_scatter(cnt, [iv], ones16)
                    for cc in range(CPT):
                        x = data_buf[buf, cc, v]
                        plsc.addupdate_scatter(accs[cc], [iv], x)

        iotaL = lax.iota(jnp.int32, _L)

        @pl.loop(0, K // _L)
        def _finalize(v):
            base = v * _L
            g0 = (base + iotaL) << 2
            c4 = (plsc.load_gather(cnt, [g0]) +
                  plsc.load_gather(cnt, [g0 + 1]) +
                  plsc.load_gather(cnt, [g0 + 2]) +
                  plsc.load_gather(cnt, [g0 + 3]))
            r = 1.0 / jnp.maximum(c4, 1.0)
            for cc in range(CPT):
                s4 = (plsc.load_gather(accs[cc], [g0]) +
                      plsc.load_gather(accs[cc], [g0 + 1]) +
                      plsc.load_gather(accs[cc], [g0 + 2]) +
                      plsc.load_gather(accs[cc], [g0 + 3]))
                accs[cc][pl.ds(base, _L)] = s4 * r

        for cc in range(CPT):
            pltpu.sync_copy(accs[cc].at[pl.ds(0, K)],
                            out_hbm.at[pl.ds((row0 + cc) * K, K)])

    return sc_pool


def kernel(img, spx):
    B, C, H, W = img.shape
    P = H * W
    K = 1024
    img2 = img.reshape(B * C, P // 16, 16)
    spx2 = spx.reshape(B, P // 16, 16)
    out = _build_sc_call(B, C, P, K)(img2, spx2)
    return out.reshape(B, C, K)
